# Initial kernel scaffold; baseline (speedup 1.0000x reference)
import jax
import jax.numpy as jnp
from jax.experimental import pallas as pl


def kernel(edge_index, vn_colors, neighbor_idx_info, vn_prefix, fn_embed, v2f_msgs, v2f_hidden, f2v_msgs, f2v_hidden, msg_prefix, color_embed, Wih_v2f, Whh_v2f, bih_v2f, bhh_v2f, Wih_f2v, Whh_f2v, bih_f2v, bhh_f2v, W1, a1s, a1d, b1, W2, a2s, a2d, b2, W3, a3s, a3d, b3, W4, a4s, a4d, b4, Wq, Wk, Ws, bs):
    raise NotImplementedError("write your pallas kernel here")


# speedup vs baseline: 1.6293x; 1.6293x over previous
import functools
import jax
import jax.numpy as jnp
from jax import lax
from jax.experimental import pallas as pl
from jax.experimental.pallas import tpu as pltpu

_NV = 50000
_NF = 512
_M = 50000
_N = _NV + _NF + 2 * _M
_LSRC = 6
_GROUP = 4


def _mm(a, b):
    return lax.dot_general(a, b, (((1,), (0,)), ((), ())),
                           preferred_element_type=jnp.float32)


def _gru_body(x_ref, h_ref, wih_ref, whh_ref, bih_ref, bhh_ref, o_ref):
    x = x_ref[...]
    h = h_ref[...]
    gi0 = _mm(x, wih_ref[0]) + bih_ref[0:1, :]
    gi1 = _mm(x, wih_ref[1]) + bih_ref[1:2, :]
    gi2 = _mm(x, wih_ref[2]) + bih_ref[2:3, :]
    gh0 = _mm(h, whh_ref[0]) + bhh_ref[0:1, :]
    gh1 = _mm(h, whh_ref[1]) + bhh_ref[1:2, :]
    gh2 = _mm(h, whh_ref[2]) + bhh_ref[2:3, :]
    r = jax.nn.sigmoid(gi0 + gh0)
    z = jax.nn.sigmoid(gi1 + gh1)
    n = jnp.tanh(gi2 + r * gh2)
    o_ref[...] = (1.0 - z) * n + z * h


def _gru(msgs, hidden, Wih, Whh, bih, bhh):
    M, DOM = msgs.shape
    H = hidden.shape[1]
    B = 512
    Mp = ((M + B - 1) // B) * B
    xp = jnp.pad(msgs, ((0, Mp - M), (0, 0)))
    hp = jnp.pad(hidden, ((0, Mp - M), (0, 128 - H)))
    wih_t = jnp.pad(Wih.reshape(3, H, DOM).transpose(0, 2, 1),
                    ((0, 0), (0, 0), (0, 128 - H)))
    whh_t = jnp.pad(Whh.reshape(3, H, H).transpose(0, 2, 1),
                    ((0, 0), (0, 128 - H), (0, 128 - H)))
    bih_p = jnp.pad(bih.reshape(3, H), ((0, 0), (0, 128 - H)))
    bhh_p = jnp.pad(bhh.reshape(3, H), ((0, 0), (0, 128 - H)))
    out = pl.pallas_call(
        _gru_body,
        grid=(Mp // B,),
        in_specs=[
            pl.BlockSpec((B, DOM), lambda i: (i, 0)),
            pl.BlockSpec((B, 128), lambda i: (i, 0)),
            pl.BlockSpec((3, DOM, 128), lambda i: (0, 0, 0)),
            pl.BlockSpec((3, 128, 128), lambda i: (0, 0, 0)),
            pl.BlockSpec((3, 128), lambda i: (0, 0)),
            pl.BlockSpec((3, 128), lambda i: (0, 0)),
        ],
        out_specs=pl.BlockSpec((B, 128), lambda i: (i, 0)),
        out_shape=jax.ShapeDtypeStruct((Mp, 128), jnp.float32),
    )(xp, hp, wih_t, whh_t, bih_p, bhh_p)
    return out[:M, :H]


def _gat(x, src, dst, W, atts, attd, bias, heads, outc, concat):
    n = x.shape[0]
    h = _mm(x, W)
    hh = h.reshape(n, heads, outc)
    a_s = (hh * atts).sum(-1)
    a_d = (hh * attd).sum(-1)
    m = jax.nn.leaky_relu(a_s.max(0) + a_d.max(0), 0.2)
    ex_self = jnp.exp(jax.nn.leaky_relu(a_s + a_d, 0.2) - m)
    alpha = jax.nn.leaky_relu(a_s[src] + a_d[dst], 0.2)
    ex = jnp.exp(alpha - m)
    den = jax.ops.segment_sum(ex, dst, num_segments=n) + ex_self
    num = jax.ops.segment_sum(hh[src] * ex[:, :, None], dst, num_segments=n)
    num = num + hh * ex_self[:, :, None]
    out = num * (1.0 / (den + 1e-16))[:, :, None]
    out = out.reshape(n, heads * outc) if concat else out.mean(axis=1)
    return out + bias


def kernel(edge_index, vn_colors, neighbor_idx_info, vn_prefix, fn_embed,
           v2f_msgs, v2f_hidden, f2v_msgs, f2v_hidden, msg_prefix,
           color_embed, Wih_v2f, Whh_v2f, bih_v2f, bhh_v2f,
           Wih_f2v, Whh_f2v, bih_f2v, bhh_f2v,
           W1, a1s, a1d, b1, W2, a2s, a2d, b2,
           W3, a3s, a3d, b3, W4, a4s, a4d, b4,
           Wq, Wk, Ws, bs):
    v2f_h = _gru(v2f_msgs, v2f_hidden, Wih_v2f, Whh_v2f, bih_v2f, bhh_v2f)
    f2v_h = _gru(f2v_msgs, f2v_hidden, Wih_f2v, Whh_f2v, bih_f2v, bhh_f2v)

    xv = jnp.concatenate([vn_prefix, color_embed[vn_colors]], axis=1)
    hidden = jnp.concatenate([v2f_h, f2v_h], axis=0)
    xm = jnp.concatenate([msg_prefix, hidden], axis=1)
    x = jnp.concatenate([xv, fn_embed, xm], axis=0)

    src = edge_index[0]
    dst = edge_index[1]
    x = jax.nn.leaky_relu(_gat(x, src, dst, W1, a1s, a1d, b1, 4, 8, True))
    x = jax.nn.leaky_relu(_gat(x, src, dst, W2, a2s, a2d, b2, 4, 8, True))
    x = jax.nn.leaky_relu(_gat(x, src, dst, W3, a3s, a3d, b3, 4, 8, True))
    x = jax.nn.leaky_relu(_gat(x, src, dst, W4, a4s, a4d, b4, 4, 32, False))

    pooling = x[_NV:_NV + _NF]
    F = _NF
    u = _mm(pooling, _mm(Wq, Ws[:32])) + bs
    v = _mm(pooling, _mm(Wk, Ws[32:]))

    tbl = neighbor_idx_info
    R = tbl.shape[0]
    idx = jnp.arange(R, dtype=jnp.int32) % F
    stride = 1 + tbl[:, 1] % max(1, F // _LSRC - 1)
    j = jnp.arange(1, _LSRC, dtype=jnp.int32)
    others = (idx[:, None] + j[None, :] * stride[:, None]) % F
    pos = tbl[:, 2] % _LSRC
    scores = jax.nn.sigmoid(u[idx][:, None, :] + v[others])
    trg = jax.nn.sigmoid(u[idx] + v[idx])
    ssum = jnp.mean(scores, axis=1)
    tot = jax.nn.softmax(jnp.stack([ssum, trg], axis=1), axis=1)
    w = jax.nn.softmax(scores, axis=1) * tot[:, 0][:, None, :] * (_GROUP - 1)
    kk = jnp.arange(_LSRC, dtype=jnp.int32)[None, :]
    gidx = jnp.clip(kk - (kk > pos[:, None]).astype(jnp.int32), 0, _LSRC - 2)
    wg = jnp.take_along_axis(w, gidx[:, :, None], axis=1)
    wfull = jnp.where((kk == pos[:, None])[:, :, None],
                      tot[:, 1][:, None, :], wg)
    return wfull.reshape(R * _LSRC, 1), v2f_h, f2v_h


# speedup vs baseline: 72.2584x; 44.3505x over previous
import functools
import jax
import jax.numpy as jnp
from jax import lax
from jax.experimental import pallas as pl
from jax.experimental.pallas import tpu as pltpu
from jax.experimental.pallas import tpu_sc as plsc

_NV = 50000
_NF = 512
_M = 50000
_N = _NV + _NF + 2 * _M
_LSRC = 6
_GROUP = 4

_NPP = 151040
_G = 150520
_CH = 128
_NCH = 147
_EP = 32 * _NCH * _CH
_BK = 64
_BR = 2360
_CAP = _NCH * _CH + _CH
_PK = 4096
_SENT = 4000
_FB = _NV // _BR
_FOFF = _NV - _FB * _BR


def _mm(a, b):
    return lax.dot_general(a, b, (((1,), (0,)), ((), ())),
                           preferred_element_type=jnp.float32)


def _lrelu(x):
    return jnp.where(x >= 0, x, 0.2 * x)


_SC_PARAMS = pltpu.CompilerParams(needs_layout_passes=False,
                                  use_tc_tiling_on_sc=False)
_MESH = dict(core_axis_name="c", subcore_axis_name="s")


def _gru_body(x_ref, h_ref, wih_ref, whh_ref, bih_ref, bhh_ref, o_ref):
    x = x_ref[...]
    h = h_ref[...]
    gi0 = _mm(x, wih_ref[0]) + bih_ref[0:1, :]
    gi1 = _mm(x, wih_ref[1]) + bih_ref[1:2, :]
    gi2 = _mm(x, wih_ref[2]) + bih_ref[2:3, :]
    gh0 = _mm(h, whh_ref[0]) + bhh_ref[0:1, :]
    gh1 = _mm(h, whh_ref[1]) + bhh_ref[1:2, :]
    gh2 = _mm(h, whh_ref[2]) + bhh_ref[2:3, :]
    r = jax.nn.sigmoid(gi0 + gh0)
    z = jax.nn.sigmoid(gi1 + gh1)
    n = jnp.tanh(gi2 + r * gh2)
    o_ref[...] = (1.0 - z) * n + z * h


def _gru(msgs, hidden, Wih, Whh, bih, bhh):
    M, DOM = msgs.shape
    H = hidden.shape[1]
    B = 512
    Mp = ((M + B - 1) // B) * B
    xp = jnp.pad(msgs, ((0, Mp - M), (0, 0)))
    hp = jnp.pad(hidden, ((0, Mp - M), (0, 128 - H)))
    wih_t = jnp.pad(Wih.reshape(3, H, DOM).transpose(0, 2, 1),
                    ((0, 0), (0, 0), (0, 128 - H)))
    whh_t = jnp.pad(Whh.reshape(3, H, H).transpose(0, 2, 1),
                    ((0, 0), (0, 128 - H), (0, 128 - H)))
    bih_p = jnp.pad(bih.reshape(3, H), ((0, 0), (0, 128 - H)))
    bhh_p = jnp.pad(bhh.reshape(3, H), ((0, 0), (0, 128 - H)))
    out = pl.pallas_call(
        _gru_body,
        grid=(Mp // B,),
        in_specs=[
            pl.BlockSpec((B, DOM), lambda i: (i, 0)),
            pl.BlockSpec((B, 128), lambda i: (i, 0)),
            pl.BlockSpec((3, DOM, 128), lambda i: (0, 0, 0)),
            pl.BlockSpec((3, 128, 128), lambda i: (0, 0, 0)),
            pl.BlockSpec((3, 128), lambda i: (0, 0)),
            pl.BlockSpec((3, 128), lambda i: (0, 0)),
        ],
        out_specs=pl.BlockSpec((B, 128), lambda i: (i, 0)),
        out_shape=jax.ShapeDtypeStruct((Mp, 128), jnp.float32),
    )(xp, hp, wih_t, whh_t, bih_p, bhh_p)
    return out[:M, :H]


def _k1_body(x_ref, w_ref, asd_ref, h_ref, sd_ref, mp_ref):
    h = _mm(x_ref[...], w_ref[...])
    sd = _mm(h, asd_ref[...])
    h_ref[...] = h
    sd_ref[...] = sd
    mp_ref[...] = jnp.max(sd, axis=0, keepdims=True)[None]


def _k1(x, W, att_s, att_d, outc):
    inc = x.shape[1]
    eye = jnp.eye(4, dtype=jnp.float32)
    as_mat = (eye[:, None, :] * att_s[:, :, None]).reshape(4 * outc, 4)
    ad_mat = (eye[:, None, :] * att_d[:, :, None]).reshape(4 * outc, 4)
    asd = jnp.concatenate(
        [as_mat, ad_mat, jnp.zeros((4 * outc, 8), jnp.float32)], axis=1)
    B = 512
    nblk = _NPP // B
    f32 = jnp.float32
    return pl.pallas_call(
        _k1_body,
        grid=(nblk,),
        in_specs=[
            pl.BlockSpec((B, inc), lambda i: (i, 0)),
            pl.BlockSpec((inc, 4 * outc), lambda i: (0, 0)),
            pl.BlockSpec((4 * outc, 16), lambda i: (0, 0)),
        ],
        out_specs=[
            pl.BlockSpec((B, 4 * outc), lambda i: (i, 0)),
            pl.BlockSpec((B, 16), lambda i: (i, 0)),
            pl.BlockSpec((1, 1, 16), lambda i: (i, 0, 0)),
        ],
        out_shape=[
            jax.ShapeDtypeStruct((_NPP, 4 * outc), f32),
            jax.ShapeDtypeStruct((_NPP, 16), f32),
            jax.ShapeDtypeStruct((nblk, 1, 16), f32),
        ],
    )(x, W, asd)


def _finalize_m(mp):
    mp = mp[:, 0, :]
    m = _lrelu(jnp.max(mp[:, :4], axis=0) + jnp.max(mp[:, 4:8], axis=0))
    return jnp.tile(m, 4), m.reshape(1, 4)


def _bucket_body(src_h, dst_h, bk_o, cnt_o, stg, srcv, dstv, cntv,
                 off_s, fl_s, sem):
    c = lax.axis_index("c")
    s = lax.axis_index("s")
    wid = c * 16 + s
    base = c * (_EP // 2) + s * (_NCH * _CH)
    it = lax.iota(jnp.int32, 16)
    dummyv = it * 0 + (_G * _PK + _SENT)

    def fill_body(f, _):
        plsc.store_scatter(stg, [it * 0 + f // 9, 16 * (f % 9) + it], dummyv)
        return 0
    lax.fori_loop(0, _BK * 9, fill_body, 0)

    def zs(b, _):
        off_s[b] = 0
        fl_s[b] = 0
        return 0
    lax.fori_loop(0, _BK, zs, 0)

    def chunk(i, _):
        pltpu.sync_copy(src_h.at[pl.ds(base + i * _CH, _CH)], srcv)
        pltpu.sync_copy(dst_h.at[pl.ds(base + i * _CH, _CH)], dstv)
        for j in range(8):
            dl = dstv[pl.ds(16 * j, 16)]
            sl = srcv[pl.ds(16 * j, 16)]
            bv = dl // _BR
            eiv = sl * _PK + (dl - bv * _BR)

            def bloop(b, _):
                msk = bv == b
                cn = jnp.sum(jnp.where(msk, 1, 0))

                @pl.when(cn > 0)
                def _():
                    off = off_s[b]
                    plsc.store_compressed(stg.at[b, pl.ds(off, 16)], eiv,
                                          mask=msk)
                    noff = off + cn

                    @pl.when(noff >= _CH)
                    def _():
                        fl = pl.multiple_of(fl_s[b], _CH)
                        pltpu.sync_copy(stg.at[b, pl.ds(0, _CH)],
                                        bk_o.at[wid, b, pl.ds(fl, _CH)])
                        tail = stg[b, pl.ds(_CH, 16)]
                        stg[b, pl.ds(0, 16)] = tail
                        fl_s[b] = fl + _CH
                        off_s[b] = noff - _CH

                    @pl.when(noff < _CH)
                    def _():
                        off_s[b] = noff
                return 0
            lax.fori_loop(0, _BK, bloop, 0)
        return 0
    lax.fori_loop(0, _NCH, chunk, 0)

    def tail(b, _):
        off = off_s[b]

        @pl.when(off > 0)
        def _():
            for k in range(8):
                plsc.store_scatter(stg.at[b], [16 * k + it], dummyv,
                                   mask=16 * k + it >= off)
            pltpu.sync_copy(
                stg.at[b, pl.ds(0, _CH)],
                bk_o.at[wid, b, pl.ds(pl.multiple_of(fl_s[b], _CH), _CH)])
            fl_s[b] = fl_s[b] + _CH
        fv = it * 0 + fl_s[b]
        plsc.store_scatter(cntv, [it * 0 + b], fv, mask=it == 0)
        return 0
    lax.fori_loop(0, _BK, tail, 0)
    pltpu.sync_copy(cntv, cnt_o.at[wid])


def _bucket_build(src_p, dst_p):
    i32 = jnp.int32
    kern = pl.kernel(
        _bucket_body,
        out_type=[
            jax.ShapeDtypeStruct((32, _BK, _CAP), i32),
            jax.ShapeDtypeStruct((32, _BK), i32),
        ],
        mesh=plsc.VectorSubcoreMesh(**_MESH),
        scratch_types=[
            pltpu.VMEM((_BK, 144), i32),
            pltpu.VMEM((_CH,), i32),
            pltpu.VMEM((_CH,), i32),
            pltpu.VMEM((_BK,), i32),
            pltpu.SMEM((_BK,), i32),
            pltpu.SMEM((_BK,), i32),
            pltpu.SemaphoreType.DMA,
        ],
        compiler_params=_SC_PARAMS,
    )
    return kern(src_p, dst_p)


def _agg_body(bk_h, cntt_h, m_h, sd_h, h_h, den_o, num_o,
              den_t, out_t, ep, srcv, dstv, rrv, sg, dg, exc, hgg,
              mv, cnt_v, sem):
    c = lax.axis_index("c")
    s = lax.axis_index("s")
    wid = c * 16 + s
    it = lax.iota(jnp.int32, 16)
    zeros16 = jnp.zeros((16,), jnp.float32)
    pltpu.sync_copy(m_h, mv)
    mval = mv[...]

    for rep in range(2):
        b = wid * 2 + rep
        nodebase = b * _BR

        def z1(i, _):
            fl = 16 * i + it
            plsc.store_scatter(den_t, [fl // 4, fl % 4], zeros16)
            return 0
        lax.fori_loop(0, (_BR + 8) * 4 // 16, z1, 0)

        def z2(i, _):
            fl = 16 * i + it
            plsc.store_scatter(out_t, [fl // 32, fl % 32], zeros16)
            return 0
        lax.fori_loop(0, (_BR + 8) * 32 // 16, z2, 0)

        pltpu.sync_copy(cntt_h.at[b], cnt_v)

        def wloop(w, _):
            nchv = plsc.load_gather(cnt_v, [it * 0 + w])
            nch = jnp.minimum(nchv[0], _CAP) // _CH

            def chunk(i, _):
                pltpu.sync_copy(
                    bk_h.at[w, b, pl.ds(pl.multiple_of(i * _CH, _CH), _CH)],
                    ep)

                def unpack(j, _):
                    p = ep[pl.ds(16 * j, 16)]
                    sl = jnp.clip(p // _PK, 0, _NPP - 1)
                    rr = p - (p // _PK) * _PK
                    rr = jnp.where((rr >= 0) & (rr < _BR), rr, _BR)
                    srcv[pl.ds(16 * j, 16)] = sl
                    dstv[pl.ds(16 * j, 16)] = nodebase + rr
                    rrv[pl.ds(16 * j, 16)] = rr
                    return 0
                lax.fori_loop(0, 8, unpack, 0, unroll=4)
                pltpu.async_copy(sd_h.at[srcv], sg, sem).wait()
                pltpu.async_copy(sd_h.at[dstv], dg, sem).wait()
                pltpu.async_copy(h_h.at[srcv], hgg, sem).wait()

                def exb(j, _):
                    fl = 16 * j + it
                    e = fl // 4
                    hd = fl % 4
                    a = (plsc.load_gather(sg, [e, hd])
                         + plsc.load_gather(dg, [e, hd + 4]))
                    a = jnp.where(a >= 0, a, 0.2 * a)
                    ex = jnp.exp(a - mval)
                    plsc.store_scatter(exc, [e, hd], ex)
                    rr = plsc.load_gather(rrv, [e])
                    plsc.addupdate_scatter(den_t, [rr, hd], ex)
                    return 0
                lax.fori_loop(0, 32, exb, 0, unroll=4)

                def wb(j, _):
                    fl = 16 * j + it
                    e = fl // 32
                    ch = fl % 32
                    hv = plsc.load_gather(hgg, [e, ch])
                    ev = plsc.load_gather(exc, [e, ch // 8])
                    rr = plsc.load_gather(rrv, [e])
                    plsc.addupdate_scatter(out_t, [rr, ch], hv * ev)
                    return 0
                lax.fori_loop(0, 64, wb, 0, unroll=4)
                return 0
            lax.fori_loop(0, nch, chunk, 0)
            return 0
        lax.fori_loop(0, 32, wloop, 0)
        pltpu.sync_copy(den_t.at[pl.ds(0, _BR)],
                        den_o.at[pl.ds(nodebase, _BR)])
        pltpu.sync_copy(out_t.at[pl.ds(0, _BR)],
                        num_o.at[pl.ds(nodebase, _BR)])


def _edge_agg(buckets, countsT, m16, SD, H32):
    f32 = jnp.float32
    i32 = jnp.int32
    kern = pl.kernel(
        _agg_body,
        out_type=[
            jax.ShapeDtypeStruct((_NPP, 4), f32),
            jax.ShapeDtypeStruct((_NPP, 32), f32),
        ],
        mesh=plsc.VectorSubcoreMesh(**_MESH),
        scratch_types=[
            pltpu.VMEM((_BR + 8, 4), f32),
            pltpu.VMEM((_BR + 8, 32), f32),
            pltpu.VMEM((_CH,), i32),
            pltpu.VMEM((_CH,), i32),
            pltpu.VMEM((_CH,), i32),
            pltpu.VMEM((_CH,), i32),
            pltpu.VMEM((_CH, 16), f32),
            pltpu.VMEM((_CH, 16), f32),
            pltpu.VMEM((_CH, 4), f32),
            pltpu.VMEM((_CH, 32), f32),
            pltpu.VMEM((16,), f32),
            pltpu.VMEM((32,), i32),
            pltpu.SemaphoreType.DMA,
        ],
        compiler_params=_SC_PARAMS,
    )
    return kern(buckets, countsT, m16, SD, H32)


def _agg4_body(bk_h, cntt_h, m_h, sd_h, h_h, den_o, num_o,
               den_t, out_t, ep, srcv, dstv, rrv, sg, dg, exc, hgg,
               mv, cnt_v, sem):
    c = lax.axis_index("c")
    s = lax.axis_index("s")
    w = c * 16 + s
    it = lax.iota(jnp.int32, 16)
    zeros16 = jnp.zeros((16,), jnp.float32)
    pltpu.sync_copy(m_h, mv)
    mval = mv[...]
    nodebase = _FB * _BR

    def z1(i, _):
        fl = 16 * i + it
        plsc.store_scatter(den_t, [fl // 4, fl % 4], zeros16)
        return 0
    lax.fori_loop(0, 520 * 4 // 16, z1, 0)

    def z2(i, _):
        fl = 16 * i + it
        plsc.store_scatter(out_t, [fl // 128, fl % 128], zeros16)
        return 0
    lax.fori_loop(0, 520 * 128 // 16, z2, 0)

    pltpu.sync_copy(cntt_h.at[_FB], cnt_v)
    nchv = plsc.load_gather(cnt_v, [it * 0 + w])
    nch = jnp.minimum(nchv[0], _CAP) // _CH

    def chunk(i, _):
        pltpu.sync_copy(
            bk_h.at[w, _FB, pl.ds(pl.multiple_of(i * _CH, _CH), _CH)], ep)

        def unpack(j, _):
            p = ep[pl.ds(16 * j, 16)]
            sl = jnp.clip(p // _PK, 0, _NPP - 1)
            rr = p - (p // _PK) * _PK
            rr4 = jnp.where((rr >= _FOFF) & (rr < _FOFF + _NF),
                            rr - _FOFF, _NF)
            srcv[pl.ds(16 * j, 16)] = sl
            dstv[pl.ds(16 * j, 16)] = nodebase + jnp.clip(rr, 0, _BR)
            rrv[pl.ds(16 * j, 16)] = rr4
            return 0
        lax.fori_loop(0, 8, unpack, 0, unroll=4)
        pltpu.async_copy(sd_h.at[srcv], sg, sem).wait()
        pltpu.async_copy(sd_h.at[dstv], dg, sem).wait()
        pltpu.async_copy(h_h.at[srcv], hgg, sem).wait()

        def exb(j, _):
            fl = 16 * j + it
            e = fl // 4
            hd = fl % 4
            a = (plsc.load_gather(sg, [e, hd])
                 + plsc.load_gather(dg, [e, hd + 4]))
            a = jnp.where(a >= 0, a, 0.2 * a)
            ex = jnp.exp(a - mval)
            plsc.store_scatter(exc, [e, hd], ex)
            rr = plsc.load_gather(rrv, [e])
            plsc.addupdate_scatter(den_t, [rr, hd], ex)
            return 0
        lax.fori_loop(0, 32, exb, 0, unroll=4)

        def wb(j, _):
            fl = 16 * j + it
            e = fl // 128
            ch = fl % 128
            hv = plsc.load_gather(hgg, [e, ch])
            ev = plsc.load_gather(exc, [e, ch // 32])
            rr = plsc.load_gather(rrv, [e])
            plsc.addupdate_scatter(out_t, [rr, ch], hv * ev)
            return 0
        lax.fori_loop(0, 256, wb, 0, unroll=4)
        return 0
    lax.fori_loop(0, nch, chunk, 0)
    pltpu.sync_copy(den_t, den_o.at[w])
    pltpu.sync_copy(out_t, num_o.at[w])


def _edge_agg4(buckets, countsT, m16, SD, H128):
    f32 = jnp.float32
    i32 = jnp.int32
    kern = pl.kernel(
        _agg4_body,
        out_type=[
            jax.ShapeDtypeStruct((32, 520, 4), f32),
            jax.ShapeDtypeStruct((32, 520, 128), f32),
        ],
        mesh=plsc.VectorSubcoreMesh(**_MESH),
        scratch_types=[
            pltpu.VMEM((520, 4), f32),
            pltpu.VMEM((520, 128), f32),
            pltpu.VMEM((_CH,), i32),
            pltpu.VMEM((_CH,), i32),
            pltpu.VMEM((_CH,), i32),
            pltpu.VMEM((_CH,), i32),
            pltpu.VMEM((_CH, 16), f32),
            pltpu.VMEM((_CH, 16), f32),
            pltpu.VMEM((_CH, 4), f32),
            pltpu.VMEM((_CH, 128), f32),
            pltpu.VMEM((16,), f32),
            pltpu.VMEM((32,), i32),
            pltpu.SemaphoreType.DMA,
        ],
        compiler_params=_SC_PARAMS,
    )
    return kern(buckets, countsT, m16, SD, H128)


def _k2_body(dp_ref, np_ref, sd_ref, m_ref, h_ref, b_ref, o_ref):
    sd = sd_ref[...]
    exs = jnp.exp(_lrelu(sd[:, :4] + sd[:, 4:8]) - m_ref[...])
    den = dp_ref[...] + exs
    inv = 1.0 / (den + 1e-16)
    h = h_ref[...]
    num = np_ref[...]
    outs = []
    for hd in range(4):
        nsum = (num[:, hd * 8:(hd + 1) * 8]
                + h[:, hd * 8:(hd + 1) * 8] * exs[:, hd:hd + 1])
        outs.append(nsum * inv[:, hd:hd + 1])
    o = jnp.concatenate(outs, axis=1) + b_ref[...]
    o_ref[...] = _lrelu(o)


def _k2(den, num, SD, m14, h, bias):
    B = 512
    nblk = _NPP // B
    return pl.pallas_call(
        _k2_body,
        grid=(nblk,),
        in_specs=[
            pl.BlockSpec((B, 4), lambda i: (i, 0)),
            pl.BlockSpec((B, 32), lambda i: (i, 0)),
            pl.BlockSpec((B, 16), lambda i: (i, 0)),
            pl.BlockSpec((1, 4), lambda i: (0, 0)),
            pl.BlockSpec((B, 32), lambda i: (i, 0)),
            pl.BlockSpec((1, 32), lambda i: (0, 0)),
        ],
        out_specs=pl.BlockSpec((B, 32), lambda i: (i, 0)),
        out_shape=jax.ShapeDtypeStruct((_NPP, 32), jnp.float32),
    )(den, num, SD, m14, h, bias)


def _k2l4_body(dp_ref, np_ref, sd_ref, m_ref, h_ref, b_ref, o_ref):
    sd = sd_ref[...]
    exs = jnp.exp(_lrelu(sd[:, :4] + sd[:, 4:8]) - m_ref[...])
    den = jnp.sum(dp_ref[...], axis=0) + exs
    inv = 1.0 / (den + 1e-16)
    num = jnp.sum(np_ref[...], axis=0)
    h = h_ref[...]
    acc = jnp.zeros((512, 32), jnp.float32)
    for hd in range(4):
        nsum = (num[:, hd * 32:(hd + 1) * 32]
                + h[:, hd * 32:(hd + 1) * 32] * exs[:, hd:hd + 1])
        acc = acc + nsum * inv[:, hd:hd + 1]
    o = acc * 0.25 + b_ref[...]
    o_ref[...] = _lrelu(o)


def _k2_l4(den, num, SD, m14, h, bias):
    return pl.pallas_call(
        _k2l4_body,
        out_shape=jax.ShapeDtypeStruct((512, 32), jnp.float32),
    )(den, num, SD, m14, h, bias)


def _gat_layer(x, buckets, countsT, W, att_s, att_d, bias):
    h, SD, mp = _k1(x, W, att_s, att_d, 8)
    m16, m14 = _finalize_m(mp)
    SDp = jnp.pad(SD, ((0, 16), (0, 0)))
    den, num = _edge_agg(buckets, countsT, m16, SDp, h)
    return _k2(den, num, SD, m14, h, bias.reshape(1, 32))


def _gat_layer4(x, buckets, countsT, W, att_s, att_d, bias):
    h, SD, mp = _k1(x, W, att_s, att_d, 32)
    m16, m14 = _finalize_m(mp)
    SDp = jnp.pad(SD, ((0, 16), (0, 0)))
    den, num = _edge_agg4(buckets, countsT, m16, SDp, h)
    lo, hi = _NV, _NV + _NF
    return _k2_l4(den[:, :_NF], num[:, :_NF],
                  SD[lo:hi], m14, h[lo:hi], bias.reshape(1, 32))


def kernel(edge_index, vn_colors, neighbor_idx_info, vn_prefix, fn_embed,
           v2f_msgs, v2f_hidden, f2v_msgs, f2v_hidden, msg_prefix,
           color_embed, Wih_v2f, Whh_v2f, bih_v2f, bhh_v2f,
           Wih_f2v, Whh_f2v, bih_f2v, bhh_f2v,
           W1, a1s, a1d, b1, W2, a2s, a2d, b2,
           W3, a3s, a3d, b3, W4, a4s, a4d, b4,
           Wq, Wk, Ws, bs):
    v2f_h = _gru(v2f_msgs, v2f_hidden, Wih_v2f, Whh_v2f, bih_v2f, bhh_v2f)
    f2v_h = _gru(f2v_msgs, f2v_hidden, Wih_f2v, Whh_f2v, bih_f2v, bhh_f2v)

    xv = jnp.concatenate([vn_prefix, color_embed[vn_colors]], axis=1)
    hidden = jnp.concatenate([v2f_h, f2v_h], axis=0)
    xm = jnp.concatenate([msg_prefix, hidden], axis=1)
    x = jnp.concatenate([xv, fn_embed, xm], axis=0)
    x = jnp.pad(x, ((0, _NPP - _N), (0, 0)))

    E = edge_index.shape[1]
    src_p = jnp.full((_EP,), _G, jnp.int32).at[:E].set(edge_index[0])
    dst_p = jnp.full((_EP,), _G, jnp.int32).at[:E].set(edge_index[1])
    buckets, counts = _bucket_build(src_p, dst_p)
    countsT = counts.T.copy()

    x = _gat_layer(x, buckets, countsT, W1, a1s, a1d, b1)
    x = _gat_layer(x, buckets, countsT, W2, a2s, a2d, b2)
    x = _gat_layer(x, buckets, countsT, W3, a3s, a3d, b3)
    pooling = _gat_layer4(x, buckets, countsT, W4, a4s, a4d, b4)

    F = _NF
    u = _mm(pooling, _mm(Wq, Ws[:32])) + bs
    v = _mm(pooling, _mm(Wk, Ws[32:]))

    tbl = neighbor_idx_info
    R = tbl.shape[0]
    idx = jnp.arange(R, dtype=jnp.int32) % F
    stride = 1 + tbl[:, 1] % max(1, F // _LSRC - 1)
    j = jnp.arange(1, _LSRC, dtype=jnp.int32)
    others = (idx[:, None] + j[None, :] * stride[:, None]) % F
    pos = tbl[:, 2] % _LSRC
    scores = jax.nn.sigmoid(u[idx][:, None, :] + v[others])
    trg = jax.nn.sigmoid(u[idx] + v[idx])
    ssum = jnp.mean(scores, axis=1)
    tot = jax.nn.softmax(jnp.stack([ssum, trg], axis=1), axis=1)
    w = jax.nn.softmax(scores, axis=1) * tot[:, 0][:, None, :] * (_GROUP - 1)
    kk = jnp.arange(_LSRC, dtype=jnp.int32)[None, :]
    gidx = jnp.clip(kk - (kk > pos[:, None]).astype(jnp.int32), 0, _LSRC - 2)
    wg = jnp.take_along_axis(w, gidx[:, :, None], axis=1)
    wfull = jnp.where((kk == pos[:, None])[:, :, None],
                      tot[:, 1][:, None, :], wg)
    return wfull.reshape(R * _LSRC, 1), v2f_h, f2v_h
